# baseline (speedup 1.0000x reference)
import functools

import jax
import jax.numpy as jnp
from jax import lax
from jax.experimental import pallas as pl
from jax.experimental.pallas import tpu as pltpu
from jax.experimental.pallas import tpu_sc as plsc

_CAPACITY_RATIO = 0.75
_LANES = 16


def _make_sc_mask(b, l, k):
    nvec = l // _LANES
    mesh = plsc.VectorSubcoreMesh(core_axis_name="c", subcore_axis_name="s")
    info = plsc.get_sparse_core_info()
    nc = info.num_cores

    def count_ge(key_ref, cand):
        def body(i, acc):
            v = key_ref[pl.ds(i * _LANES, _LANES)]
            return acc + jnp.where(v >= cand, 1, 0).astype(jnp.int32)
        acc = lax.fori_loop(0, nvec, body, jnp.zeros((_LANES,), jnp.int32))
        return _splat_sum(acc)

    @functools.partial(
        pl.kernel, mesh=mesh,
        out_type=jax.ShapeDtypeStruct((b, l), jnp.float32),
        scratch_types=[
            pltpu.VMEM((l,), jnp.float32),
            pltpu.VMEM((l,), jnp.uint32),
            pltpu.VMEM((l,), jnp.float32),
        ],
    )
    def mask_sc(scores_hbm, mask_hbm, row_v, key_v, out_v):
        wid = lax.axis_index("s") * nc + lax.axis_index("c")

        @pl.when(wid < b)
        def _():
            pltpu.sync_copy(scores_hbm.at[wid], row_v)

            def to_key(i, carry):
                u = lax.bitcast_convert_type(
                    row_v[pl.ds(i * _LANES, _LANES)], jnp.uint32)
                neg = (u >> jnp.uint32(31)) > jnp.uint32(0)
                key = jnp.where(neg, u ^ jnp.uint32(0xFFFFFFFF),
                                u | jnp.uint32(0x80000000))
                key_v[pl.ds(i * _LANES, _LANES)] = key
                return carry
            lax.fori_loop(0, nvec, to_key, 0)

            tau = jnp.zeros((_LANES,), jnp.uint32)
            for bit in range(31, -1, -1):
                cand = tau | jnp.uint32(1 << bit)
                tau = jnp.where(count_ge(key_v, cand) >= k, cand, tau)

            cnt_gt = count_ge(key_v, tau) - _count_eq(key_v, tau, nvec)
            need = k - cnt_gt

            tsel = jnp.zeros((_LANES,), jnp.int32)
            for bit in range(12, -1, -1):
                cand_t = tsel | jnp.int32(1 << bit)

                def body_t(i, acc):
                    v = key_v[pl.ds(i * _LANES, _LANES)]
                    idx = lax.iota(jnp.int32, _LANES) + i * _LANES
                    hit = (v == tau) & (idx < cand_t)
                    return acc + jnp.where(hit, 1, 0).astype(jnp.int32)
                g = _splat_sum(lax.fori_loop(
                    0, nvec, body_t, jnp.zeros((_LANES,), jnp.int32)))
                tsel = jnp.where(g < need, cand_t, tsel)
            tstar = tsel + 1

            def write(i, carry):
                v = key_v[pl.ds(i * _LANES, _LANES)]
                idx = lax.iota(jnp.int32, _LANES) + i * _LANES
                sel = (v > tau) | ((v == tau) & (idx < tstar))
                out_v[pl.ds(i * _LANES, _LANES)] = jnp.where(sel, 1.0, 0.0)
                return carry
            lax.fori_loop(0, nvec, write, 0)

            pltpu.sync_copy(out_v, mask_hbm.at[wid])

    return mask_sc


def _splat_sum(v):
    idx = lax.iota(jnp.int32, _LANES)
    for sh in (8, 4, 2, 1):
        perm = (idx + sh) & (_LANES - 1)
        v = v + v.at[perm].get(mode="promise_in_bounds")
    return v


def _count_eq(key_ref, tau, nvec):
    def body(i, acc):
        v = key_ref[pl.ds(i * _LANES, _LANES)]
        return acc + jnp.where(v == tau, 1, 0).astype(jnp.int32)
    acc = lax.fori_loop(0, nvec, body, jnp.zeros((_LANES,), jnp.int32))
    return _splat_sum(acc)


def _mask_kernel(scores_ref, mask_ref, *, k):
    s = scores_ref[...]
    b, l = s.shape
    u = jax.lax.bitcast_convert_type(s, jnp.uint32)
    neg = (u >> 31).astype(jnp.bool_)
    key = jnp.where(neg, ~u, u | jnp.uint32(0x80000000))
    tau = jnp.zeros((b, 1), jnp.uint32)
    for bit in range(31, -1, -1):
        cand = tau | jnp.uint32(1 << bit)
        cnt = jnp.sum((key >= cand).astype(jnp.int32), axis=1, keepdims=True)
        tau = jnp.where(cnt >= k, cand, tau)
    gt = key > tau
    eq = key == tau
    cnt_gt = jnp.sum(gt.astype(jnp.int32), axis=1, keepdims=True)
    need = k - cnt_gt
    idx = jax.lax.broadcasted_iota(jnp.int32, (b, l), 1)
    t = jnp.zeros((b, 1), jnp.int32)
    for bit in range(12, -1, -1):
        cand = t | (1 << bit)
        g = jnp.sum((eq & (idx < cand)).astype(jnp.int32), axis=1, keepdims=True)
        t = jnp.where(g < need, cand, t)
    sel = gt | (eq & (idx < (t + 1)))
    mask_ref[...] = sel.astype(jnp.float32)


def _mlp_kernel(x_ref, mask_ref, w1_ref, w2_ref, out_ref, *, nchunk):
    xb = x_ref[...]
    m = mask_ref[...]
    x16 = xb.astype(jnp.bfloat16)
    ff = w1_ref.shape[1]
    tf = ff // nchunk
    y = None
    for c in range(nchunk):
        h = jnp.dot(x16, w1_ref[:, c * tf:(c + 1) * tf],
                    preferred_element_type=jnp.float32)
        h = jnp.maximum(h, 0.0).astype(jnp.bfloat16)
        yc = jnp.dot(h, w2_ref[c * tf:(c + 1) * tf, :],
                     preferred_element_type=jnp.float32)
        y = yc if y is None else y + yc
    out_ref[...] = jnp.where(m > 0.0, y, xb)


def kernel(x, w_router, W1, W2):
    b, l, d = x.shape
    ff = W1.shape[1]
    k = max(1, int(l * _CAPACITY_RATIO))
    scores = jnp.einsum('bld,d->bl', x, w_router)
    if k >= l:
        mask = jnp.ones((b, l), jnp.float32)
    else:
        mask = _make_sc_mask(b, l, k)(scores)

    xf = x.reshape(b * l, d)
    maskf = mask.reshape(b * l, 1)
    tm = min(512, b * l)
    grid = (b * l // tm,)
    out = pl.pallas_call(
        functools.partial(_mlp_kernel, nchunk=4),
        grid=grid,
        in_specs=[
            pl.BlockSpec((tm, d), lambda i: (i, 0)),
            pl.BlockSpec((tm, 1), lambda i: (i, 0)),
            pl.BlockSpec((d, ff), lambda i: (0, 0)),
            pl.BlockSpec((ff, d), lambda i: (0, 0)),
        ],
        out_specs=pl.BlockSpec((tm, d), lambda i: (i, 0)),
        out_shape=jax.ShapeDtypeStruct((b * l, d), jnp.float32),
        compiler_params=pltpu.CompilerParams(
            dimension_semantics=("parallel",),
            vmem_limit_bytes=110 * 1024 * 1024,
        ),
    )(xf, maskf, W1.astype(jnp.bfloat16), W2.astype(jnp.bfloat16))
    return out.reshape(b, l, d)


# speedup vs baseline: 1.0230x; 1.0230x over previous
import functools

import jax
import jax.numpy as jnp
from jax import lax
from jax.experimental import pallas as pl
from jax.experimental.pallas import tpu as pltpu
from jax.experimental.pallas import tpu_sc as plsc

_CAPACITY_RATIO = 0.75
_LANES = 16


def _make_sc_mask(b, l, k):
    nvec = l // _LANES
    mesh = plsc.VectorSubcoreMesh(core_axis_name="c", subcore_axis_name="s")
    info = plsc.get_sparse_core_info()
    nc = info.num_cores

    unroll = 8

    def count_ge(key_ref, cand):
        def body(i, acc):
            for u in range(unroll):
                v = key_ref[pl.ds(i * (_LANES * unroll) + u * _LANES, _LANES)]
                acc = acc + jnp.where(v >= cand, 1, 0).astype(jnp.int32)
            return acc
        acc = lax.fori_loop(0, nvec // unroll, body,
                            jnp.zeros((_LANES,), jnp.int32))
        return _splat_sum(acc)

    @functools.partial(
        pl.kernel, mesh=mesh,
        out_type=jax.ShapeDtypeStruct((b, l), jnp.float32),
        scratch_types=[
            pltpu.VMEM((l,), jnp.float32),
            pltpu.VMEM((l,), jnp.uint32),
            pltpu.VMEM((l,), jnp.float32),
        ],
    )
    def mask_sc(scores_hbm, mask_hbm, row_v, key_v, out_v):
        wid = lax.axis_index("s") * nc + lax.axis_index("c")

        @pl.when(wid < b)
        def _():
            pltpu.sync_copy(scores_hbm.at[wid], row_v)

            def to_key(i, carry):
                for u_ in range(unroll):
                    sl = pl.ds(i * (_LANES * unroll) + u_ * _LANES, _LANES)
                    u = lax.bitcast_convert_type(row_v[sl], jnp.uint32)
                    neg = (u >> jnp.uint32(31)) > jnp.uint32(0)
                    key_v[sl] = jnp.where(neg, u ^ jnp.uint32(0xFFFFFFFF),
                                          u | jnp.uint32(0x80000000))
                return carry
            lax.fori_loop(0, nvec // unroll, to_key, 0)

            tau = jnp.zeros((_LANES,), jnp.uint32)
            for bit in range(31, -1, -1):
                cand = tau | jnp.uint32(1 << bit)
                tau = jnp.where(count_ge(key_v, cand) >= k, cand, tau)

            cnt_gt = count_ge(key_v, tau) - _count_eq(key_v, tau, nvec)
            need = k - cnt_gt

            tsel = jnp.zeros((_LANES,), jnp.int32)
            for bit in range(12, -1, -1):
                cand_t = tsel | jnp.int32(1 << bit)

                def body_t(i, acc):
                    for u_ in range(unroll):
                        off = i * (_LANES * unroll) + u_ * _LANES
                        v = key_v[pl.ds(off, _LANES)]
                        idx = lax.iota(jnp.int32, _LANES) + off
                        hit = (v == tau) & (idx < cand_t)
                        acc = acc + jnp.where(hit, 1, 0).astype(jnp.int32)
                    return acc
                g = _splat_sum(lax.fori_loop(
                    0, nvec // unroll, body_t,
                    jnp.zeros((_LANES,), jnp.int32)))
                tsel = jnp.where(g < need, cand_t, tsel)
            tstar = tsel + 1

            def write(i, carry):
                for u_ in range(unroll):
                    off = i * (_LANES * unroll) + u_ * _LANES
                    v = key_v[pl.ds(off, _LANES)]
                    idx = lax.iota(jnp.int32, _LANES) + off
                    sel = (v > tau) | ((v == tau) & (idx < tstar))
                    out_v[pl.ds(off, _LANES)] = jnp.where(sel, 1.0, 0.0)
                return carry
            lax.fori_loop(0, nvec // unroll, write, 0)

            pltpu.sync_copy(out_v, mask_hbm.at[wid])

    return mask_sc


def _splat_sum(v):
    idx = lax.iota(jnp.int32, _LANES)
    for sh in (8, 4, 2, 1):
        perm = (idx + sh) & (_LANES - 1)
        v = v + v.at[perm].get(mode="promise_in_bounds")
    return v


def _count_eq(key_ref, tau, nvec, unroll=8):
    def body(i, acc):
        for u in range(unroll):
            v = key_ref[pl.ds(i * (_LANES * unroll) + u * _LANES, _LANES)]
            acc = acc + jnp.where(v == tau, 1, 0).astype(jnp.int32)
        return acc
    acc = lax.fori_loop(0, nvec // unroll, body,
                        jnp.zeros((_LANES,), jnp.int32))
    return _splat_sum(acc)


def _mask_kernel(scores_ref, mask_ref, *, k):
    s = scores_ref[...]
    b, l = s.shape
    u = jax.lax.bitcast_convert_type(s, jnp.uint32)
    neg = (u >> 31).astype(jnp.bool_)
    key = jnp.where(neg, ~u, u | jnp.uint32(0x80000000))
    tau = jnp.zeros((b, 1), jnp.uint32)
    for bit in range(31, -1, -1):
        cand = tau | jnp.uint32(1 << bit)
        cnt = jnp.sum((key >= cand).astype(jnp.int32), axis=1, keepdims=True)
        tau = jnp.where(cnt >= k, cand, tau)
    gt = key > tau
    eq = key == tau
    cnt_gt = jnp.sum(gt.astype(jnp.int32), axis=1, keepdims=True)
    need = k - cnt_gt
    idx = jax.lax.broadcasted_iota(jnp.int32, (b, l), 1)
    t = jnp.zeros((b, 1), jnp.int32)
    for bit in range(12, -1, -1):
        cand = t | (1 << bit)
        g = jnp.sum((eq & (idx < cand)).astype(jnp.int32), axis=1, keepdims=True)
        t = jnp.where(g < need, cand, t)
    sel = gt | (eq & (idx < (t + 1)))
    mask_ref[...] = sel.astype(jnp.float32)


def _mlp_kernel(x_ref, mask_ref, w1_ref, w2_ref, out_ref, *, nchunk):
    xb = x_ref[...]
    m = mask_ref[...]
    x16 = xb.astype(jnp.bfloat16)
    ff = w1_ref.shape[1]
    tf = ff // nchunk
    y = None
    for c in range(nchunk):
        h = jnp.dot(x16, w1_ref[:, c * tf:(c + 1) * tf],
                    preferred_element_type=jnp.float32)
        h = jnp.maximum(h, 0.0).astype(jnp.bfloat16)
        yc = jnp.dot(h, w2_ref[c * tf:(c + 1) * tf, :],
                     preferred_element_type=jnp.float32)
        y = yc if y is None else y + yc
    out_ref[...] = jnp.where(m > 0.0, y, xb)


def kernel(x, w_router, W1, W2):
    b, l, d = x.shape
    ff = W1.shape[1]
    k = max(1, int(l * _CAPACITY_RATIO))
    scores = jnp.einsum('bld,d->bl', x, w_router)
    if k >= l:
        mask = jnp.ones((b, l), jnp.float32)
    else:
        mask = _make_sc_mask(b, l, k)(scores)

    xf = x.reshape(b * l, d)
    maskf = mask.reshape(b * l, 1)
    tm = min(512, b * l)
    grid = (b * l // tm,)
    out = pl.pallas_call(
        functools.partial(_mlp_kernel, nchunk=4),
        grid=grid,
        in_specs=[
            pl.BlockSpec((tm, d), lambda i: (i, 0)),
            pl.BlockSpec((tm, 1), lambda i: (i, 0)),
            pl.BlockSpec((d, ff), lambda i: (0, 0)),
            pl.BlockSpec((ff, d), lambda i: (0, 0)),
        ],
        out_specs=pl.BlockSpec((tm, d), lambda i: (i, 0)),
        out_shape=jax.ShapeDtypeStruct((b * l, d), jnp.float32),
        compiler_params=pltpu.CompilerParams(
            dimension_semantics=("parallel",),
            vmem_limit_bytes=110 * 1024 * 1024,
        ),
    )(xf, maskf, W1.astype(jnp.bfloat16), W2.astype(jnp.bfloat16))
    return out.reshape(b, l, d)


# speedup vs baseline: 1.0263x; 1.0032x over previous
import functools

import jax
import jax.numpy as jnp
from jax import lax
from jax.experimental import pallas as pl
from jax.experimental.pallas import tpu as pltpu
from jax.experimental.pallas import tpu_sc as plsc

_CAPACITY_RATIO = 0.75
_LANES = 16


def _make_sc_mask(b, l, k):
    nvec = l // _LANES
    mesh = plsc.VectorSubcoreMesh(core_axis_name="c", subcore_axis_name="s")
    info = plsc.get_sparse_core_info()
    nc = info.num_cores

    unroll = 8

    def count_ge(key_ref, cand):
        def body(i, acc):
            for u in range(unroll):
                v = key_ref[pl.ds(i * (_LANES * unroll) + u * _LANES, _LANES)]
                acc = acc + jnp.where(v >= cand, 1, 0).astype(jnp.int32)
            return acc
        acc = lax.fori_loop(0, nvec // unroll, body,
                            jnp.zeros((_LANES,), jnp.int32))
        return _splat_sum(acc)

    @functools.partial(
        pl.kernel, mesh=mesh,
        out_type=jax.ShapeDtypeStruct((b, l), jnp.float32),
        scratch_types=[
            pltpu.VMEM((l,), jnp.float32),
            pltpu.VMEM((l,), jnp.uint32),
            pltpu.VMEM((l,), jnp.float32),
        ],
    )
    def mask_sc(scores_hbm, mask_hbm, row_v, key_v, out_v):
        wid = lax.axis_index("s") * nc + lax.axis_index("c")

        @pl.when(wid < b)
        def _():
            pltpu.sync_copy(scores_hbm.at[wid], row_v)

            def to_key(i, carry):
                for u_ in range(unroll):
                    sl = pl.ds(i * (_LANES * unroll) + u_ * _LANES, _LANES)
                    u = lax.bitcast_convert_type(row_v[sl], jnp.uint32)
                    neg = (u >> jnp.uint32(31)) > jnp.uint32(0)
                    key_v[sl] = jnp.where(neg, u ^ jnp.uint32(0xFFFFFFFF),
                                          u | jnp.uint32(0x80000000))
                return carry
            lax.fori_loop(0, nvec // unroll, to_key, 0)

            tau = jnp.zeros((_LANES,), jnp.uint32)
            for bit in range(31, -1, -1):
                cand = tau | jnp.uint32(1 << bit)
                tau = jnp.where(count_ge(key_v, cand) >= k, cand, tau)

            cnt_gt = count_ge(key_v, tau) - _count_eq(key_v, tau, nvec)
            need = k - cnt_gt

            tsel = jnp.zeros((_LANES,), jnp.int32)
            for bit in range(12, -1, -1):
                cand_t = tsel | jnp.int32(1 << bit)

                def body_t(i, acc):
                    for u_ in range(unroll):
                        off = i * (_LANES * unroll) + u_ * _LANES
                        v = key_v[pl.ds(off, _LANES)]
                        idx = lax.iota(jnp.int32, _LANES) + off
                        hit = (v == tau) & (idx < cand_t)
                        acc = acc + jnp.where(hit, 1, 0).astype(jnp.int32)
                    return acc
                g = _splat_sum(lax.fori_loop(
                    0, nvec // unroll, body_t,
                    jnp.zeros((_LANES,), jnp.int32)))
                tsel = jnp.where(g < need, cand_t, tsel)
            tstar = tsel + 1

            def write(i, carry):
                for u_ in range(unroll):
                    off = i * (_LANES * unroll) + u_ * _LANES
                    v = key_v[pl.ds(off, _LANES)]
                    idx = lax.iota(jnp.int32, _LANES) + off
                    sel = (v > tau) | ((v == tau) & (idx < tstar))
                    out_v[pl.ds(off, _LANES)] = jnp.where(sel, 1.0, 0.0)
                return carry
            lax.fori_loop(0, nvec // unroll, write, 0)

            pltpu.sync_copy(out_v, mask_hbm.at[wid])

    return mask_sc


def _splat_sum(v):
    idx = lax.iota(jnp.int32, _LANES)
    for sh in (8, 4, 2, 1):
        perm = (idx + sh) & (_LANES - 1)
        v = v + v.at[perm].get(mode="promise_in_bounds")
    return v


def _count_eq(key_ref, tau, nvec, unroll=8):
    def body(i, acc):
        for u in range(unroll):
            v = key_ref[pl.ds(i * (_LANES * unroll) + u * _LANES, _LANES)]
            acc = acc + jnp.where(v == tau, 1, 0).astype(jnp.int32)
        return acc
    acc = lax.fori_loop(0, nvec // unroll, body,
                        jnp.zeros((_LANES,), jnp.int32))
    return _splat_sum(acc)


def _mask_kernel(scores_ref, mask_ref, *, k):
    s = scores_ref[...]
    b, l = s.shape
    u = jax.lax.bitcast_convert_type(s, jnp.uint32)
    neg = (u >> 31).astype(jnp.bool_)
    key = jnp.where(neg, ~u, u | jnp.uint32(0x80000000))
    tau = jnp.zeros((b, 1), jnp.uint32)
    for bit in range(31, -1, -1):
        cand = tau | jnp.uint32(1 << bit)
        cnt = jnp.sum((key >= cand).astype(jnp.int32), axis=1, keepdims=True)
        tau = jnp.where(cnt >= k, cand, tau)
    gt = key > tau
    eq = key == tau
    cnt_gt = jnp.sum(gt.astype(jnp.int32), axis=1, keepdims=True)
    need = k - cnt_gt
    idx = jax.lax.broadcasted_iota(jnp.int32, (b, l), 1)
    t = jnp.zeros((b, 1), jnp.int32)
    for bit in range(12, -1, -1):
        cand = t | (1 << bit)
        g = jnp.sum((eq & (idx < cand)).astype(jnp.int32), axis=1, keepdims=True)
        t = jnp.where(g < need, cand, t)
    sel = gt | (eq & (idx < (t + 1)))
    mask_ref[...] = sel.astype(jnp.float32)


def _mlp_kernel(x_ref, mask_ref, w1_ref, w2_ref, out_ref, *, nchunk):
    xb = x_ref[...]
    m = mask_ref[...]
    x16 = xb.astype(jnp.bfloat16)
    ff = w1_ref.shape[1]
    tf = ff // nchunk
    y = None
    for c in range(nchunk):
        h = jnp.dot(x16, w1_ref[:, c * tf:(c + 1) * tf],
                    preferred_element_type=jnp.float32)
        h = jnp.maximum(h, 0.0).astype(jnp.bfloat16)
        yc = jnp.dot(h, w2_ref[c * tf:(c + 1) * tf, :],
                     preferred_element_type=jnp.float32)
        y = yc if y is None else y + yc
    out_ref[...] = jnp.where(m > 0.0, y, xb)


def kernel(x, w_router, W1, W2):
    b, l, d = x.shape
    ff = W1.shape[1]
    k = max(1, int(l * _CAPACITY_RATIO))
    scores = jnp.einsum('bld,d->bl', x, w_router)
    if k >= l:
        mask = jnp.ones((b, l), jnp.float32)
    else:
        mask = _make_sc_mask(b, l, k)(scores)

    xf = x.reshape(b * l, d)
    maskf = mask.reshape(b * l, 1)
    tm = min(512, b * l)
    grid = (b * l // tm,)
    out = pl.pallas_call(
        functools.partial(_mlp_kernel, nchunk=1),
        grid=grid,
        in_specs=[
            pl.BlockSpec((tm, d), lambda i: (i, 0)),
            pl.BlockSpec((tm, 1), lambda i: (i, 0)),
            pl.BlockSpec((d, ff), lambda i: (0, 0)),
            pl.BlockSpec((ff, d), lambda i: (0, 0)),
        ],
        out_specs=pl.BlockSpec((tm, d), lambda i: (i, 0)),
        out_shape=jax.ShapeDtypeStruct((b * l, d), jnp.float32),
        compiler_params=pltpu.CompilerParams(
            dimension_semantics=("parallel",),
            vmem_limit_bytes=110 * 1024 * 1024,
        ),
    )(xf, maskf, W1.astype(jnp.bfloat16), W2.astype(jnp.bfloat16))
    return out.reshape(b, l, d)
